# baseline (device time: 16835 ns/iter reference)
import jax
import jax.numpy as jnp
from jax import lax
from jax.experimental import pallas as pl
from jax.experimental.pallas import tpu as pltpu

N_Y = 4


def kernel(x):
    m, n = x.shape
    blk = n // N_Y

    def body(x_ref, out_ref, ready, send_sems, recv_sems):
        my_x = lax.axis_index("x")
        my_y = lax.axis_index("y")
        my_z = lax.axis_index("z")

        barrier_sem = pltpu.get_barrier_semaphore()
        pl.semaphore_signal(barrier_sem, inc=1)
        pl.semaphore_wait(barrier_sem, 1)

        for d in range(1, N_Y):
            s = lax.rem(my_y + N_Y - d, N_Y)
            pl.semaphore_signal(
                ready.at[d - 1], inc=1,
                device_id=(my_x, s, my_z),
                device_id_type=pl.DeviceIdType.MESH,
            )

        rdmas = []
        for dy in range(1, N_Y):
            j = lax.rem(my_y + dy, N_Y)
            pl.semaphore_wait(ready.at[dy - 1], 1)
            rdma = pltpu.make_async_remote_copy(
                src_ref=x_ref.at[:, pl.ds(j * blk, blk)],
                dst_ref=out_ref.at[pl.ds(my_y * m, m), :],
                send_sem=send_sems.at[dy - 1],
                recv_sem=recv_sems.at[dy - 1],
                device_id=(my_x, j, my_z),
                device_id_type=pl.DeviceIdType.MESH,
            )
            rdma.start()
            rdmas.append(rdma)

        out_ref[pl.ds(my_y * m, m), :] = x_ref[:, pl.ds(my_y * blk, blk)]

        for rdma in rdmas:
            rdma.wait()

    return pl.pallas_call(
        body,
        out_shape=jax.ShapeDtypeStruct((N_Y * m, blk), x.dtype),
        in_specs=[pl.BlockSpec(memory_space=pltpu.VMEM)],
        out_specs=pl.BlockSpec(memory_space=pltpu.VMEM),
        scratch_shapes=[
            pltpu.SemaphoreType.REGULAR((N_Y - 1,)),
            pltpu.SemaphoreType.DMA((N_Y - 1,)),
            pltpu.SemaphoreType.DMA((N_Y - 1,)),
        ],
        compiler_params=pltpu.CompilerParams(collective_id=0),
    )(x)
